# baseline (speedup 1.0000x reference)
import functools

import jax
import jax.numpy as jnp
from jax import lax
from jax.experimental import pallas as pl
from jax.experimental.pallas import tpu as pltpu
from jax.experimental.pallas import tpu_sc as plsc

_N_NODES = 50000
_N_GRAPHS = 512
_N_SPECIES = 119
_NW = 16
_CHUNK = 3136
_TAIL = _N_NODES - (_NW - 1) * _CHUNK
_EXTRA = _CHUNK - _TAIL
_ROW = 529
_LANES = 16
_ACC = 8512
_GPW = _N_GRAPHS // _NW

_SOFTPLUS_C = 0.5413248546129181


def _ln_1to2(x):
    z = (x - 1.0) / (x + 1.0)
    z2 = z * z
    return 2.0 * z * (1.0 + z2 * (1.0 / 3.0 + z2 * (0.2 + z2 * (1.0 / 7.0 + z2 * (1.0 / 9.0)))))


def _softplus(v):
    t = jnp.exp(-jnp.abs(v))
    return jnp.maximum(v, 0.0) + _ln_1to2(1.0 + t)


def _sc_body(e_hbm, sp_hbm, g_hbm, scale_hbm, shift_hbm, gs_hbm, gsh_hbm, nn_hbm,
             out_hbm,
             e_v, sp_v, g_v, scale_v, shift_v, gs_v, gsh_v, nn_v,
             acc_sc, acc_sh, acc_e, res_sc, res_sh, res_e,
             gat_v, fin_v, shared, sem_in, sem_out):
    wid = lax.axis_index("s")
    base = wid * _CHUNK
    not_last = wid != _NW - 1

    pltpu.async_copy(e_hbm.at[pl.ds(base, _TAIL)], e_v.at[pl.ds(0, _TAIL)], sem_in)
    pltpu.async_copy(sp_hbm.at[pl.ds(base, _TAIL)], sp_v.at[pl.ds(0, _TAIL)], sem_in)
    pltpu.async_copy(g_hbm.at[pl.ds(base, _TAIL)], g_v.at[pl.ds(0, _TAIL)], sem_in)
    pltpu.async_copy(scale_hbm, scale_v, sem_in)
    pltpu.async_copy(shift_hbm, shift_v, sem_in)
    pltpu.async_copy(gs_hbm, gs_v.at[pl.ds(0, 1)], sem_in)
    pltpu.async_copy(gsh_hbm, gsh_v.at[pl.ds(0, 1)], sem_in)
    pltpu.async_copy(nn_hbm, nn_v, sem_in)

    @pl.when(not_last)
    def _fire_extra():
        pltpu.async_copy(e_hbm.at[pl.ds(base + _TAIL, _EXTRA)],
                         e_v.at[pl.ds(_TAIL, _EXTRA)], sem_in)
        pltpu.async_copy(sp_hbm.at[pl.ds(base + _TAIL, _EXTRA)],
                         sp_v.at[pl.ds(_TAIL, _EXTRA)], sem_in)
        pltpu.async_copy(g_hbm.at[pl.ds(base + _TAIL, _EXTRA)],
                         g_v.at[pl.ds(_TAIL, _EXTRA)], sem_in)

    zero = jnp.zeros((_LANES,), jnp.float32)

    @plsc.parallel_loop(0, _ACC // _LANES, unroll=4)
    def _zero(j):
        sl = pl.ds(j * _LANES, _LANES)
        acc_sc[sl] = zero
        acc_sh[sl] = zero
        acc_e[sl] = zero

    pltpu.make_async_copy(e_hbm.at[pl.ds(base, _TAIL)], e_v.at[pl.ds(0, _TAIL)], sem_in).wait()
    pltpu.make_async_copy(sp_hbm.at[pl.ds(base, _TAIL)], sp_v.at[pl.ds(0, _TAIL)], sem_in).wait()
    pltpu.make_async_copy(g_hbm.at[pl.ds(base, _TAIL)], g_v.at[pl.ds(0, _TAIL)], sem_in).wait()
    pltpu.make_async_copy(scale_hbm, scale_v, sem_in).wait()
    pltpu.make_async_copy(shift_hbm, shift_v, sem_in).wait()
    pltpu.make_async_copy(gs_hbm, gs_v.at[pl.ds(0, 1)], sem_in).wait()
    pltpu.make_async_copy(gsh_hbm, gsh_v.at[pl.ds(0, 1)], sem_in).wait()
    pltpu.make_async_copy(nn_hbm, nn_v, sem_in).wait()

    @pl.when(not_last)
    def _drain_extra():
        pltpu.make_async_copy(e_hbm.at[pl.ds(base + _TAIL, _EXTRA)],
                              e_v.at[pl.ds(_TAIL, _EXTRA)], sem_in).wait()
        pltpu.make_async_copy(sp_hbm.at[pl.ds(base + _TAIL, _EXTRA)],
                              sp_v.at[pl.ds(_TAIL, _EXTRA)], sem_in).wait()
        pltpu.make_async_copy(g_hbm.at[pl.ds(base + _TAIL, _EXTRA)],
                              g_v.at[pl.ds(_TAIL, _EXTRA)], sem_in).wait()

    lane_off = lax.iota(jnp.int32, _LANES) * _ROW

    def do_group(gi):
        sl = pl.ds(gi * _LANES, _LANES)
        addr = lane_off + g_v[sl]
        plsc.addupdate_scatter(acc_sc, [addr], plsc.load_gather(scale_v, [sp_v[sl]]))
        plsc.addupdate_scatter(acc_sh, [addr], plsc.load_gather(shift_v, [sp_v[sl]]))
        plsc.addupdate_scatter(acc_e, [addr], e_v[sl])

    g_tail = _TAIL // _LANES
    g_all = _CHUNK // _LANES

    def step4(i, carry):
        for k in range(4):
            do_group(i * 4 + k)
        return carry

    lax.fori_loop(0, g_tail // 4, step4, 0)
    for gi in range(4 * (g_tail // 4), g_tail):
        do_group(gi)

    @pl.when(not_last)
    def _steps_extra():
        def step4b(i, carry):
            for k in range(4):
                do_group(g_tail + i * 4 + k)
            return carry

        n4 = (g_all - g_tail) // 4
        lax.fori_loop(0, n4, step4b, 0)
        for gi in range(g_tail + 4 * n4, g_all):
            do_group(gi)

    for q, (acc, res) in enumerate(
        ((acc_sc, res_sc), (acc_sh, res_sh), (acc_e, res_e))
    ):
        @plsc.parallel_loop(0, _N_GRAPHS // _LANES, unroll=2)
        def _reduce(j, acc=acc, res=res):
            off = j * _LANES
            vals = [acc[pl.ds(lane * _ROW + off, _LANES)] for lane in range(_LANES)]
            while len(vals) > 1:
                vals = [a + b for a, b in zip(vals[::2], vals[1::2])]
            res[pl.ds(off, _LANES)] = vals[0]

        pltpu.sync_copy(res, shared.at[q, wid])

    plsc.subcore_barrier()

    col = wid * _GPW
    lane0 = lax.iota(jnp.int32, _LANES) == 0
    ones = jnp.ones((_LANES,), jnp.float32)
    gs = jnp.sum(jnp.where(lane0, gs_v[pl.ds(0, _LANES)], 0.0)) * ones
    gsh = jnp.sum(jnp.where(lane0, gsh_v[pl.ds(0, _LANES)], 0.0)) * ones
    tgs = _softplus(gs + _SOFTPLUS_C)

    sums = []
    for q in range(3):
        for w2 in range(_NW):
            pltpu.async_copy(shared.at[q, w2, pl.ds(col, _GPW)],
                             gat_v.at[q, w2], sem_in)
    for q in range(3):
        for w2 in range(_NW):
            pltpu.make_async_copy(shared.at[q, w2, pl.ds(col, _GPW)],
                                  gat_v.at[q, w2], sem_in).wait()
    for q in range(3):
        vecs = []
        for j in range(_GPW // _LANES):
            vals = [gat_v[q, lane, pl.ds(j * _LANES, _LANES)] for lane in range(_NW)]
            while len(vals) > 1:
                vals = [a + b for a, b in zip(vals[::2], vals[1::2])]
            vecs.append(vals[0])
        sums.append(vecs)

    for j in range(_GPW // _LANES):
        nn = nn_v[pl.ds(col + j * _LANES, _LANES)]
        na = jnp.maximum(nn.astype(jnp.float32), 1.0)
        sc = _softplus(sums[0][j] + _SOFTPLUS_C) / na * tgs
        sh = sums[1][j] / na + gsh
        fin_v[pl.ds(j * _LANES, _LANES)] = (sums[2][j] / na) * sc + sh

    pltpu.async_copy(fin_v, out_hbm.at[pl.ds(col, _GPW)], sem_out)
    pltpu.make_async_copy(fin_v, out_hbm.at[pl.ds(col, _GPW)], sem_out).wait()


@functools.cache
def _build_sc():
    mesh = plsc.VectorSubcoreMesh(
        core_axis_name="c", subcore_axis_name="s", num_cores=1
    )
    return pl.kernel(
        _sc_body,
        out_type=jax.ShapeDtypeStruct((_N_GRAPHS,), jnp.float32),
        mesh=mesh,
        compiler_params=pltpu.CompilerParams(
            needs_layout_passes=False, use_tc_tiling_on_sc=False
        ),
        scratch_types=[
            pltpu.VMEM((_CHUNK,), jnp.float32),
            pltpu.VMEM((_CHUNK,), jnp.int32),
            pltpu.VMEM((_CHUNK,), jnp.int32),
            pltpu.VMEM((_N_SPECIES,), jnp.float32),
            pltpu.VMEM((_N_SPECIES,), jnp.float32),
            pltpu.VMEM((_LANES,), jnp.float32),
            pltpu.VMEM((_LANES,), jnp.float32),
            pltpu.VMEM((_N_GRAPHS,), jnp.int32),
            pltpu.VMEM((_ACC,), jnp.float32),
            pltpu.VMEM((_ACC,), jnp.float32),
            pltpu.VMEM((_ACC,), jnp.float32),
            pltpu.VMEM((_N_GRAPHS,), jnp.float32),
            pltpu.VMEM((_N_GRAPHS,), jnp.float32),
            pltpu.VMEM((_N_GRAPHS,), jnp.float32),
            pltpu.VMEM((3, _NW, _GPW), jnp.float32),
            pltpu.VMEM((_GPW,), jnp.float32),
            pltpu.VMEM_SHARED((3, _NW, _N_GRAPHS), jnp.float32),
            pltpu.SemaphoreType.DMA,
            pltpu.SemaphoreType.DMA,
        ],
    )


def kernel(energies, scale, shift, global_scale, global_shift, species, graph_i, n_node):
    out = _build_sc()(
        energies, species.astype(jnp.int32), graph_i.astype(jnp.int32),
        scale, shift, global_scale, global_shift, n_node.astype(jnp.int32),
    )
    return out[:, None]


# speedup vs baseline: 1.0008x; 1.0008x over previous
import functools

import jax
import jax.numpy as jnp
from jax import lax
from jax.experimental import pallas as pl
from jax.experimental.pallas import tpu as pltpu
from jax.experimental.pallas import tpu_sc as plsc

_N_NODES = 50000
_N_GRAPHS = 512
_N_SPECIES = 119
_NW = 16
_CHUNK = 3136
_TAIL = _N_NODES - (_NW - 1) * _CHUNK
_EXTRA = _CHUNK - _TAIL
_ROW = 529
_LANES = 16
_TROW = 129
_ACC = 8512
_GPW = _N_GRAPHS // _NW

_SOFTPLUS_C = 0.5413248546129181


def _ln_1to2(x):
    z = (x - 1.0) / (x + 1.0)
    z2 = z * z
    return 2.0 * z * (1.0 + z2 * (1.0 / 3.0 + z2 * (0.2 + z2 * (1.0 / 7.0 + z2 * (1.0 / 9.0)))))


def _softplus(v):
    t = jnp.exp(-jnp.abs(v))
    return jnp.maximum(v, 0.0) + _ln_1to2(1.0 + t)


def _sc_body(e_hbm, sp_hbm, g_hbm, scale_hbm, shift_hbm, gs_hbm, gsh_hbm, nn_hbm,
             out_hbm,
             e_v, sp_v, g_v, scale_v, shift_v, scale_r, shift_r, gs_v, gsh_v, nn_v,
             acc_sc, acc_sh, acc_e, res_sc, res_sh, res_e,
             gat_v, fin_v, shared, sem_in, sem_out):
    wid = lax.axis_index("s")
    base = wid * _CHUNK
    not_last = wid != _NW - 1

    pltpu.async_copy(e_hbm.at[pl.ds(base, _TAIL)], e_v.at[pl.ds(0, _TAIL)], sem_in)
    pltpu.async_copy(sp_hbm.at[pl.ds(base, _TAIL)], sp_v.at[pl.ds(0, _TAIL)], sem_in)
    pltpu.async_copy(g_hbm.at[pl.ds(base, _TAIL)], g_v.at[pl.ds(0, _TAIL)], sem_in)
    pltpu.async_copy(scale_hbm, scale_v, sem_in)
    pltpu.async_copy(shift_hbm, shift_v, sem_in)
    pltpu.async_copy(gs_hbm, gs_v.at[pl.ds(0, 1)], sem_in)
    pltpu.async_copy(gsh_hbm, gsh_v.at[pl.ds(0, 1)], sem_in)
    pltpu.async_copy(nn_hbm, nn_v, sem_in)

    @pl.when(not_last)
    def _fire_extra():
        pltpu.async_copy(e_hbm.at[pl.ds(base + _TAIL, _EXTRA)],
                         e_v.at[pl.ds(_TAIL, _EXTRA)], sem_in)
        pltpu.async_copy(sp_hbm.at[pl.ds(base + _TAIL, _EXTRA)],
                         sp_v.at[pl.ds(_TAIL, _EXTRA)], sem_in)
        pltpu.async_copy(g_hbm.at[pl.ds(base + _TAIL, _EXTRA)],
                         g_v.at[pl.ds(_TAIL, _EXTRA)], sem_in)

    zero = jnp.zeros((_LANES,), jnp.float32)

    @plsc.parallel_loop(0, _ACC // _LANES, unroll=4)
    def _zero(j):
        sl = pl.ds(j * _LANES, _LANES)
        acc_sc[sl] = zero
        acc_sh[sl] = zero
        acc_e[sl] = zero

    pltpu.make_async_copy(e_hbm.at[pl.ds(base, _TAIL)], e_v.at[pl.ds(0, _TAIL)], sem_in).wait()
    pltpu.make_async_copy(sp_hbm.at[pl.ds(base, _TAIL)], sp_v.at[pl.ds(0, _TAIL)], sem_in).wait()
    pltpu.make_async_copy(g_hbm.at[pl.ds(base, _TAIL)], g_v.at[pl.ds(0, _TAIL)], sem_in).wait()
    pltpu.make_async_copy(scale_hbm, scale_v, sem_in).wait()
    pltpu.make_async_copy(shift_hbm, shift_v, sem_in).wait()
    pltpu.make_async_copy(gs_hbm, gs_v.at[pl.ds(0, 1)], sem_in).wait()
    pltpu.make_async_copy(gsh_hbm, gsh_v.at[pl.ds(0, 1)], sem_in).wait()
    pltpu.make_async_copy(nn_hbm, nn_v, sem_in).wait()

    @pl.when(not_last)
    def _drain_extra():
        pltpu.make_async_copy(e_hbm.at[pl.ds(base + _TAIL, _EXTRA)],
                              e_v.at[pl.ds(_TAIL, _EXTRA)], sem_in).wait()
        pltpu.make_async_copy(sp_hbm.at[pl.ds(base + _TAIL, _EXTRA)],
                              sp_v.at[pl.ds(_TAIL, _EXTRA)], sem_in).wait()
        pltpu.make_async_copy(g_hbm.at[pl.ds(base + _TAIL, _EXTRA)],
                              g_v.at[pl.ds(_TAIL, _EXTRA)], sem_in).wait()

    for tbl, rep in ((scale_v, scale_r), (shift_v, shift_r)):
        tv = [tbl[pl.ds(k * _LANES, _LANES)] for k in range(7)]
        tv.append(tbl[pl.ds(_N_SPECIES - _LANES, _LANES)])
        for lane in range(_LANES):
            for k in range(7):
                rep[pl.ds(lane * _TROW + k * _LANES, _LANES)] = tv[k]
            rep[pl.ds(lane * _TROW + _N_SPECIES - _LANES, _LANES)] = tv[7]

    lane_off = lax.iota(jnp.int32, _LANES) * _ROW
    lane_off_t = lax.iota(jnp.int32, _LANES) * _TROW

    def do_group(gi):
        sl = pl.ds(gi * _LANES, _LANES)
        addr = lane_off + g_v[sl]
        addr_t = lane_off_t + sp_v[sl]
        plsc.addupdate_scatter(acc_sc, [addr], plsc.load_gather(scale_r, [addr_t]))
        plsc.addupdate_scatter(acc_sh, [addr], plsc.load_gather(shift_r, [addr_t]))
        plsc.addupdate_scatter(acc_e, [addr], e_v[sl])

    g_tail = _TAIL // _LANES
    g_all = _CHUNK // _LANES

    def step4(i, carry):
        for k in range(4):
            do_group(i * 4 + k)
        return carry

    lax.fori_loop(0, g_tail // 4, step4, 0)
    for gi in range(4 * (g_tail // 4), g_tail):
        do_group(gi)

    @pl.when(not_last)
    def _steps_extra():
        def step4b(i, carry):
            for k in range(4):
                do_group(g_tail + i * 4 + k)
            return carry

        n4 = (g_all - g_tail) // 4
        lax.fori_loop(0, n4, step4b, 0)
        for gi in range(g_tail + 4 * n4, g_all):
            do_group(gi)

    for q, (acc, res) in enumerate(
        ((acc_sc, res_sc), (acc_sh, res_sh), (acc_e, res_e))
    ):
        @plsc.parallel_loop(0, _N_GRAPHS // _LANES, unroll=2)
        def _reduce(j, acc=acc, res=res):
            off = j * _LANES
            vals = [acc[pl.ds(lane * _ROW + off, _LANES)] for lane in range(_LANES)]
            while len(vals) > 1:
                vals = [a + b for a, b in zip(vals[::2], vals[1::2])]
            res[pl.ds(off, _LANES)] = vals[0]

        pltpu.sync_copy(res, shared.at[q, wid])

    plsc.subcore_barrier()

    col = wid * _GPW
    lane0 = lax.iota(jnp.int32, _LANES) == 0
    ones = jnp.ones((_LANES,), jnp.float32)
    gs = jnp.sum(jnp.where(lane0, gs_v[pl.ds(0, _LANES)], 0.0)) * ones
    gsh = jnp.sum(jnp.where(lane0, gsh_v[pl.ds(0, _LANES)], 0.0)) * ones
    tgs = _softplus(gs + _SOFTPLUS_C)

    sums = []
    for q in range(3):
        for w2 in range(_NW):
            pltpu.async_copy(shared.at[q, w2, pl.ds(col, _GPW)],
                             gat_v.at[q, w2], sem_in)
    for q in range(3):
        for w2 in range(_NW):
            pltpu.make_async_copy(shared.at[q, w2, pl.ds(col, _GPW)],
                                  gat_v.at[q, w2], sem_in).wait()
    for q in range(3):
        vecs = []
        for j in range(_GPW // _LANES):
            vals = [gat_v[q, lane, pl.ds(j * _LANES, _LANES)] for lane in range(_NW)]
            while len(vals) > 1:
                vals = [a + b for a, b in zip(vals[::2], vals[1::2])]
            vecs.append(vals[0])
        sums.append(vecs)

    for j in range(_GPW // _LANES):
        nn = nn_v[pl.ds(col + j * _LANES, _LANES)]
        na = jnp.maximum(nn.astype(jnp.float32), 1.0)
        sc = _softplus(sums[0][j] + _SOFTPLUS_C) / na * tgs
        sh = sums[1][j] / na + gsh
        fin_v[pl.ds(j * _LANES, _LANES)] = (sums[2][j] / na) * sc + sh

    pltpu.async_copy(fin_v, out_hbm.at[pl.ds(col, _GPW)], sem_out)
    pltpu.make_async_copy(fin_v, out_hbm.at[pl.ds(col, _GPW)], sem_out).wait()


@functools.cache
def _build_sc():
    mesh = plsc.VectorSubcoreMesh(
        core_axis_name="c", subcore_axis_name="s", num_cores=1
    )
    return pl.kernel(
        _sc_body,
        out_type=jax.ShapeDtypeStruct((_N_GRAPHS,), jnp.float32),
        mesh=mesh,
        compiler_params=pltpu.CompilerParams(
            needs_layout_passes=False, use_tc_tiling_on_sc=False
        ),
        scratch_types=[
            pltpu.VMEM((_CHUNK,), jnp.float32),
            pltpu.VMEM((_CHUNK,), jnp.int32),
            pltpu.VMEM((_CHUNK,), jnp.int32),
            pltpu.VMEM((_N_SPECIES,), jnp.float32),
            pltpu.VMEM((_N_SPECIES,), jnp.float32),
            pltpu.VMEM((_LANES * _TROW,), jnp.float32),
            pltpu.VMEM((_LANES * _TROW,), jnp.float32),
            pltpu.VMEM((_LANES,), jnp.float32),
            pltpu.VMEM((_LANES,), jnp.float32),
            pltpu.VMEM((_N_GRAPHS,), jnp.int32),
            pltpu.VMEM((_ACC,), jnp.float32),
            pltpu.VMEM((_ACC,), jnp.float32),
            pltpu.VMEM((_ACC,), jnp.float32),
            pltpu.VMEM((_N_GRAPHS,), jnp.float32),
            pltpu.VMEM((_N_GRAPHS,), jnp.float32),
            pltpu.VMEM((_N_GRAPHS,), jnp.float32),
            pltpu.VMEM((3, _NW, _GPW), jnp.float32),
            pltpu.VMEM((_GPW,), jnp.float32),
            pltpu.VMEM_SHARED((3, _NW, _N_GRAPHS), jnp.float32),
            pltpu.SemaphoreType.DMA,
            pltpu.SemaphoreType.DMA,
        ],
    )


def kernel(energies, scale, shift, global_scale, global_shift, species, graph_i, n_node):
    out = _build_sc()(
        energies, species.astype(jnp.int32), graph_i.astype(jnp.int32),
        scale, shift, global_scale, global_shift, n_node.astype(jnp.int32),
    )
    return out[:, None]


# speedup vs baseline: 1.1313x; 1.1303x over previous
import functools

import jax
import jax.numpy as jnp
from jax import lax
from jax.experimental import pallas as pl
from jax.experimental.pallas import tpu as pltpu
from jax.experimental.pallas import tpu_sc as plsc

_N_NODES = 50000
_N_GRAPHS = 512
_N_SPECIES = 119
_NW = 16
_CHUNK = 3136
_TAIL = _N_NODES - (_NW - 1) * _CHUNK
_EXTRA = _CHUNK - _TAIL
_ROW = 529
_LANES = 16
_ACC = 8512
_GPW = _N_GRAPHS // _NW

_SOFTPLUS_C = 0.5413248546129181


def _ln_1to2(x):
    z = (x - 1.0) / (x + 1.0)
    z2 = z * z
    return 2.0 * z * (1.0 + z2 * (1.0 / 3.0 + z2 * (0.2 + z2 * (1.0 / 7.0 + z2 * (1.0 / 9.0)))))


def _softplus(v):
    t = jnp.exp(-jnp.abs(v))
    return jnp.maximum(v, 0.0) + _ln_1to2(1.0 + t)


def _sc_body(e_hbm, sp_hbm, g_hbm, scale_hbm, shift_hbm, gs_hbm, gsh_hbm, nn_hbm,
             out_hbm,
             e_v, sp_v, g_v, scale_v, shift_v, gs_v, gsh_v, nn_v,
             acc_sc, acc_sh, acc_e, res_sc, res_sh, res_e,
             gat_v, fin_v, shared, sem_in, sem_out):
    wid = lax.axis_index("s")
    base = wid * _CHUNK
    not_last = wid != _NW - 1

    pltpu.async_copy(e_hbm.at[pl.ds(base, _TAIL)], e_v.at[pl.ds(0, _TAIL)], sem_in)
    pltpu.async_copy(sp_hbm.at[pl.ds(base, _TAIL)], sp_v.at[pl.ds(0, _TAIL)], sem_in)
    pltpu.async_copy(g_hbm.at[pl.ds(base, _TAIL)], g_v.at[pl.ds(0, _TAIL)], sem_in)
    pltpu.async_copy(scale_hbm, scale_v, sem_in)
    pltpu.async_copy(shift_hbm, shift_v, sem_in)
    pltpu.async_copy(gs_hbm, gs_v.at[pl.ds(0, 1)], sem_in)
    pltpu.async_copy(gsh_hbm, gsh_v.at[pl.ds(0, 1)], sem_in)
    pltpu.async_copy(nn_hbm, nn_v, sem_in)

    @pl.when(not_last)
    def _fire_extra():
        pltpu.async_copy(e_hbm.at[pl.ds(base + _TAIL, _EXTRA)],
                         e_v.at[pl.ds(_TAIL, _EXTRA)], sem_in)
        pltpu.async_copy(sp_hbm.at[pl.ds(base + _TAIL, _EXTRA)],
                         sp_v.at[pl.ds(_TAIL, _EXTRA)], sem_in)
        pltpu.async_copy(g_hbm.at[pl.ds(base + _TAIL, _EXTRA)],
                         g_v.at[pl.ds(_TAIL, _EXTRA)], sem_in)

    zero = jnp.zeros((_LANES,), jnp.float32)

    @plsc.parallel_loop(0, _ACC // _LANES, unroll=4)
    def _zero(j):
        sl = pl.ds(j * _LANES, _LANES)
        acc_sc[sl] = zero
        acc_sh[sl] = zero
        acc_e[sl] = zero

    pltpu.make_async_copy(e_hbm.at[pl.ds(base, _TAIL)], e_v.at[pl.ds(0, _TAIL)], sem_in).wait()
    pltpu.make_async_copy(sp_hbm.at[pl.ds(base, _TAIL)], sp_v.at[pl.ds(0, _TAIL)], sem_in).wait()
    pltpu.make_async_copy(g_hbm.at[pl.ds(base, _TAIL)], g_v.at[pl.ds(0, _TAIL)], sem_in).wait()
    pltpu.make_async_copy(scale_hbm, scale_v, sem_in).wait()
    pltpu.make_async_copy(shift_hbm, shift_v, sem_in).wait()
    pltpu.make_async_copy(gs_hbm, gs_v.at[pl.ds(0, 1)], sem_in).wait()
    pltpu.make_async_copy(gsh_hbm, gsh_v.at[pl.ds(0, 1)], sem_in).wait()
    pltpu.make_async_copy(nn_hbm, nn_v, sem_in).wait()

    @pl.when(not_last)
    def _drain_extra():
        pltpu.make_async_copy(e_hbm.at[pl.ds(base + _TAIL, _EXTRA)],
                              e_v.at[pl.ds(_TAIL, _EXTRA)], sem_in).wait()
        pltpu.make_async_copy(sp_hbm.at[pl.ds(base + _TAIL, _EXTRA)],
                              sp_v.at[pl.ds(_TAIL, _EXTRA)], sem_in).wait()
        pltpu.make_async_copy(g_hbm.at[pl.ds(base + _TAIL, _EXTRA)],
                              g_v.at[pl.ds(_TAIL, _EXTRA)], sem_in).wait()

    lane_off = lax.iota(jnp.int32, _LANES) * _ROW

    def do_group(gi):
        sl = pl.ds(gi * _LANES, _LANES)
        addr = lane_off + g_v[sl]
        plsc.addupdate_scatter(acc_sc, [addr], plsc.load_gather(scale_v, [sp_v[sl]]))
        plsc.addupdate_scatter(acc_sh, [addr], plsc.load_gather(shift_v, [sp_v[sl]]))
        plsc.addupdate_scatter(acc_e, [addr], e_v[sl])

    g_tail = _TAIL // _LANES
    g_all = _CHUNK // _LANES

    @plsc.parallel_loop(0, g_tail - 1, unroll=4)
    def _main(gi):
        do_group(gi)

    do_group(g_tail - 1)

    @pl.when(not_last)
    def _steps_extra():
        @plsc.parallel_loop(g_tail, g_all, unroll=1)
        def _main_extra(gi):
            do_group(gi)

    for q, (acc, res) in enumerate(
        ((acc_sc, res_sc), (acc_sh, res_sh), (acc_e, res_e))
    ):
        @plsc.parallel_loop(0, _N_GRAPHS // _LANES, unroll=2)
        def _reduce(j, acc=acc, res=res):
            off = j * _LANES
            vals = [acc[pl.ds(lane * _ROW + off, _LANES)] for lane in range(_LANES)]
            while len(vals) > 1:
                vals = [a + b for a, b in zip(vals[::2], vals[1::2])]
            res[pl.ds(off, _LANES)] = vals[0]

        pltpu.sync_copy(res, shared.at[q, wid])

    plsc.subcore_barrier()

    col = wid * _GPW
    lane0 = lax.iota(jnp.int32, _LANES) == 0
    ones = jnp.ones((_LANES,), jnp.float32)
    gs = jnp.sum(jnp.where(lane0, gs_v[pl.ds(0, _LANES)], 0.0)) * ones
    gsh = jnp.sum(jnp.where(lane0, gsh_v[pl.ds(0, _LANES)], 0.0)) * ones
    tgs = _softplus(gs + _SOFTPLUS_C)

    sums = []
    for q in range(3):
        for w2 in range(_NW):
            pltpu.async_copy(shared.at[q, w2, pl.ds(col, _GPW)],
                             gat_v.at[q, w2], sem_in)
    for q in range(3):
        for w2 in range(_NW):
            pltpu.make_async_copy(shared.at[q, w2, pl.ds(col, _GPW)],
                                  gat_v.at[q, w2], sem_in).wait()
    for q in range(3):
        vecs = []
        for j in range(_GPW // _LANES):
            vals = [gat_v[q, lane, pl.ds(j * _LANES, _LANES)] for lane in range(_NW)]
            while len(vals) > 1:
                vals = [a + b for a, b in zip(vals[::2], vals[1::2])]
            vecs.append(vals[0])
        sums.append(vecs)

    for j in range(_GPW // _LANES):
        nn = nn_v[pl.ds(col + j * _LANES, _LANES)]
        na = jnp.maximum(nn.astype(jnp.float32), 1.0)
        sc = _softplus(sums[0][j] + _SOFTPLUS_C) / na * tgs
        sh = sums[1][j] / na + gsh
        fin_v[pl.ds(j * _LANES, _LANES)] = (sums[2][j] / na) * sc + sh

    pltpu.async_copy(fin_v, out_hbm.at[pl.ds(col, _GPW)], sem_out)
    pltpu.make_async_copy(fin_v, out_hbm.at[pl.ds(col, _GPW)], sem_out).wait()


@functools.cache
def _build_sc():
    mesh = plsc.VectorSubcoreMesh(
        core_axis_name="c", subcore_axis_name="s", num_cores=1
    )
    return pl.kernel(
        _sc_body,
        out_type=jax.ShapeDtypeStruct((_N_GRAPHS,), jnp.float32),
        mesh=mesh,
        compiler_params=pltpu.CompilerParams(
            needs_layout_passes=False, use_tc_tiling_on_sc=False
        ),
        scratch_types=[
            pltpu.VMEM((_CHUNK,), jnp.float32),
            pltpu.VMEM((_CHUNK,), jnp.int32),
            pltpu.VMEM((_CHUNK,), jnp.int32),
            pltpu.VMEM((_N_SPECIES,), jnp.float32),
            pltpu.VMEM((_N_SPECIES,), jnp.float32),
            pltpu.VMEM((_LANES,), jnp.float32),
            pltpu.VMEM((_LANES,), jnp.float32),
            pltpu.VMEM((_N_GRAPHS,), jnp.int32),
            pltpu.VMEM((_ACC,), jnp.float32),
            pltpu.VMEM((_ACC,), jnp.float32),
            pltpu.VMEM((_ACC,), jnp.float32),
            pltpu.VMEM((_N_GRAPHS,), jnp.float32),
            pltpu.VMEM((_N_GRAPHS,), jnp.float32),
            pltpu.VMEM((_N_GRAPHS,), jnp.float32),
            pltpu.VMEM((3, _NW, _GPW), jnp.float32),
            pltpu.VMEM((_GPW,), jnp.float32),
            pltpu.VMEM_SHARED((3, _NW, _N_GRAPHS), jnp.float32),
            pltpu.SemaphoreType.DMA,
            pltpu.SemaphoreType.DMA,
        ],
    )


def kernel(energies, scale, shift, global_scale, global_shift, species, graph_i, n_node):
    out = _build_sc()(
        energies, species.astype(jnp.int32), graph_i.astype(jnp.int32),
        scale, shift, global_scale, global_shift, n_node.astype(jnp.int32),
    )
    return out[:, None]
